# baseline (speedup 1.0000x reference)
import functools

import jax
import jax.numpy as jnp
from jax import lax
from jax.experimental import pallas as pl
from jax.experimental.pallas import tpu as pltpu
from jax.experimental.pallas import tpu_sc as plsc

EMBSIZE = 64
EPS = 1e-5

NUM_CORES = 2
NUM_SUBCORES = 16
NW = NUM_CORES * NUM_SUBCORES

CHUNK = 128


def _rsqrt(x):
    i = lax.bitcast_convert_type(x, jnp.int32)
    y = lax.bitcast_convert_type(jnp.int32(0x5F3759DF) - (i >> 1), jnp.float32)
    half = jnp.float32(0.5) * x
    for _ in range(3):
        y = y * (jnp.float32(1.5) - half * y * y)
    return y


def _make_kernel(n_rows):
    rows_per_w = n_rows // NW
    n_chunks = rows_per_w // CHUNK
    mesh = plsc.VectorSubcoreMesh(core_axis_name="c", subcore_axis_name="s")

    @functools.partial(
        pl.kernel,
        out_type=jax.ShapeDtypeStruct((n_rows, EMBSIZE), jnp.float32),
        mesh=mesh,
        scratch_types=[
            pltpu.VMEM((n_chunks, CHUNK), jnp.int32),
            pltpu.VMEM((CHUNK, EMBSIZE), jnp.float32),
            pltpu.VMEM((EMBSIZE,), jnp.float32),
            pltpu.VMEM((EMBSIZE,), jnp.float32),
            pltpu.SemaphoreType.DMA,
        ],
        compiler_params=pltpu.CompilerParams(use_tc_tiling_on_sc=False),
    )
    def k(idx_hbm, table_hbm, gamma_hbm, beta_hbm, out_hbm,
          idx_v, rows_v, gamma_v, beta_v, sem):
        wid = lax.axis_index("s") * NUM_CORES + lax.axis_index("c")
        pltpu.sync_copy(idx_hbm.at[wid], idx_v)
        pltpu.sync_copy(gamma_hbm, gamma_v)
        pltpu.sync_copy(beta_hbm, beta_v)

        g = [gamma_v[pl.ds(16 * t, 16)] for t in range(4)]
        b = [beta_v[pl.ds(16 * t, 16)] for t in range(4)]
        inv_n = jnp.float32(1.0 / EMBSIZE)
        lane = lax.iota(jnp.int32, 16)
        perms = [lane ^ jnp.int32(sh) for sh in (8, 4, 2, 1)]

        dnums = lax.GatherDimensionNumbers(
            offset_dims=(), collapsed_slice_dims=(0,), start_index_map=(0,))

        def xlane_sum(v):
            for p in perms:
                v = v + lax.gather(
                    v, p[:, None], dnums, slice_sizes=(1,),
                    mode=lax.GatherScatterMode.PROMISE_IN_BOUNDS)
            return v

        def row_body(r, carry):
            v = [rows_v[r, pl.ds(16 * t, 16)] for t in range(4)]
            s = (v[0] + v[1]) + (v[2] + v[3])
            sq = (v[0] * v[0] + v[1] * v[1]) + (v[2] * v[2] + v[3] * v[3])
            mean_v = xlane_sum(s) * inv_n
            var_v = xlane_sum(sq) * inv_n - mean_v * mean_v
            istd_v = _rsqrt(var_v + jnp.float32(EPS))
            for t in range(4):
                rows_v[r, pl.ds(16 * t, 16)] = (
                    (v[t] - mean_v) * istd_v * g[t] + b[t])
            return carry

        def chunk_body(j, carry):
            pltpu.async_copy(table_hbm.at[idx_v.at[j]], rows_v, sem).wait()
            lax.fori_loop(0, CHUNK, row_body, 0, unroll=4)
            base = pl.multiple_of(wid * rows_per_w + j * CHUNK, CHUNK)
            pltpu.sync_copy(rows_v, out_hbm.at[pl.ds(base, CHUNK)])
            return carry

        lax.fori_loop(0, n_chunks, chunk_body, 0)

    return k


@jax.jit
def kernel(x, table, ln_gamma, ln_beta):
    batch, hist = x.shape
    n_rows = batch * hist
    idx = x.reshape(NW, (n_rows // NW) // CHUNK, CHUNK)
    out = _make_kernel(n_rows)(idx, table, ln_gamma, ln_beta)
    return out.reshape(batch, hist, EMBSIZE)


# speedup vs baseline: 1.1995x; 1.1995x over previous
import functools

import jax
import jax.numpy as jnp
from jax import lax
from jax.experimental import pallas as pl
from jax.experimental.pallas import tpu as pltpu
from jax.experimental.pallas import tpu_sc as plsc

EMBSIZE = 64
EPS = 1e-5

NUM_CORES = 2
NUM_SUBCORES = 16
NW = NUM_CORES * NUM_SUBCORES

CHUNK = 128


def _rsqrt(x):
    i = lax.bitcast_convert_type(x, jnp.int32)
    y = lax.bitcast_convert_type(jnp.int32(0x5F3759DF) - (i >> 1), jnp.float32)
    half = jnp.float32(0.5) * x
    for _ in range(3):
        y = y * (jnp.float32(1.5) - half * y * y)
    return y


def _make_kernel(n_rows):
    rows_per_w = n_rows // NW
    n_chunks = rows_per_w // CHUNK
    mesh = plsc.VectorSubcoreMesh(core_axis_name="c", subcore_axis_name="s")

    @functools.partial(
        pl.kernel,
        out_type=jax.ShapeDtypeStruct((n_rows, EMBSIZE), jnp.float32),
        mesh=mesh,
        scratch_types=[
            pltpu.VMEM((n_chunks, CHUNK), jnp.int32),
            pltpu.VMEM((CHUNK, EMBSIZE), jnp.float32),
            pltpu.VMEM((CHUNK, EMBSIZE), jnp.float32),
            pltpu.VMEM((CHUNK, EMBSIZE), jnp.float32),
            pltpu.VMEM((CHUNK, EMBSIZE), jnp.float32),
            pltpu.VMEM((EMBSIZE,), jnp.float32),
            pltpu.VMEM((EMBSIZE,), jnp.float32),
            pltpu.SemaphoreType.DMA,
            pltpu.SemaphoreType.DMA,
            pltpu.SemaphoreType.DMA,
            pltpu.SemaphoreType.DMA,
            pltpu.SemaphoreType.DMA,
            pltpu.SemaphoreType.DMA,
            pltpu.SemaphoreType.DMA,
            pltpu.SemaphoreType.DMA,
        ],
        compiler_params=pltpu.CompilerParams(use_tc_tiling_on_sc=False),
    )
    def k(idx_hbm, table_hbm, gamma_hbm, beta_hbm, out_hbm,
          idx_v, rows0, rows1, rows2, rows3, gamma_v, beta_v,
          g0, g1, g2, g3, s0, s1, s2, s3):
        wid = lax.axis_index("s") * NUM_CORES + lax.axis_index("c")
        bufs = [rows0, rows1, rows2, rows3]
        gsem = [g0, g1, g2, g3]
        ssem = [s0, s1, s2, s3]
        pltpu.sync_copy(idx_hbm.at[wid], idx_v)
        pltpu.sync_copy(gamma_hbm, gamma_v)
        pltpu.sync_copy(beta_hbm, beta_v)

        g = [gamma_v[pl.ds(16 * t, 16)] for t in range(4)]
        b = [beta_v[pl.ds(16 * t, 16)] for t in range(4)]
        inv_n = jnp.float32(1.0 / EMBSIZE)
        lane = lax.iota(jnp.int32, 16)
        perms = [lane ^ jnp.int32(sh) for sh in (8, 4, 2, 1)]

        dnums = lax.GatherDimensionNumbers(
            offset_dims=(), collapsed_slice_dims=(0,), start_index_map=(0,))

        def xlane_sum(v):
            for p in perms:
                v = v + lax.gather(
                    v, p[:, None], dnums, slice_sizes=(1,),
                    mode=lax.GatherScatterMode.PROMISE_IN_BOUNDS)
            return v

        def compute(buf):
            def row_body(r, carry):
                v = [buf[r, pl.ds(16 * t, 16)] for t in range(4)]
                s = (v[0] + v[1]) + (v[2] + v[3])
                sq = (v[0] * v[0] + v[1] * v[1]) + (v[2] * v[2] + v[3] * v[3])
                mean_v = xlane_sum(s) * inv_n
                var_v = xlane_sum(sq) * inv_n - mean_v * mean_v
                istd_v = _rsqrt(var_v + jnp.float32(EPS))
                for t in range(4):
                    buf[r, pl.ds(16 * t, 16)] = (
                        (v[t] - mean_v) * istd_v * g[t] + b[t])
                return carry
            lax.fori_loop(0, CHUNK, row_body, 0, unroll=4)

        def out_slice(c):
            base = pl.multiple_of(wid * rows_per_w + c * CHUNK, CHUNK)
            return out_hbm.at[pl.ds(base, CHUNK)]

        def start_gather(c, t):
            pltpu.async_copy(table_hbm.at[idx_v.at[c]], bufs[t], gsem[t])

        def wait_gather(c, t):
            pltpu.make_async_copy(
                table_hbm.at[idx_v.at[c]], bufs[t], gsem[t]).wait()

        def start_store(c, t):
            pltpu.async_copy(bufs[t], out_slice(c), ssem[t])

        def wait_store(c, t):
            pltpu.make_async_copy(bufs[t], out_slice(c), ssem[t]).wait()

        start_gather(0, 0)
        start_gather(1, 1)

        def quad_body(i, carry):
            for t in range(4):
                c = i * 4 + t
                tp = (t + 2) % 4
                @pl.when(c + 2 < n_chunks)
                def _():
                    @pl.when(c >= 2)
                    def _():
                        wait_store(c - 2, tp)
                    start_gather(c + 2, tp)
                wait_gather(c, t)
                compute(bufs[t])
                start_store(c, t)
            return carry

        lax.fori_loop(0, n_chunks // 4, quad_body, 0)
        for t in range(4):
            wait_store(n_chunks - 4 + t, t)

    return k


@jax.jit
def kernel(x, table, ln_gamma, ln_beta):
    batch, hist = x.shape
    n_rows = batch * hist
    idx = x.reshape(NW, (n_rows // NW) // CHUNK, CHUNK)
    out = _make_kernel(n_rows)(idx, table, ln_gamma, ln_beta)
    return out.reshape(batch, hist, EMBSIZE)
